# Initial kernel scaffold; baseline (speedup 1.0000x reference)
import jax
import jax.numpy as jnp
from jax.experimental import pallas as pl


def kernel(x, edge_index, feat_edge_index, conv_w, conv_b, fc_w, fc_b, lin1_w, lin1_b, gru_w_ih, gru_w_hh, gru_b_ih, gru_b_hh, gcn1_w, gcn1_b, gcn2_w, gcn2_b, mlp_w, mlp_b):
    raise NotImplementedError("write your pallas kernel here")


# speedup vs baseline: 27.5878x; 27.5878x over previous
import functools

import jax
import jax.numpy as jnp
from jax import lax
from jax.experimental import pallas as pl
from jax.experimental.pallas import tpu as pltpu
from jax.experimental.pallas import tpu_sc as plsc

N = 10000
E = 640000
EB = 80
KJ = 5
ROWS = E // EB
D_PAD = 48
DEG_W = 16
NC = 2
NS = 16
NW = NC * NS
ROWS_W = ROWS // NW
NIT = ROWS_W // KJ
ROWS_T = N // NS
BN = 400


def _sc_mesh():
    return plsc.VectorSubcoreMesh(core_axis_name="c", subcore_axis_name="s")


_SC_PARAMS = pltpu.CompilerParams(use_tc_tiling_on_sc=False)


def _deg_partials(d1, d2, ones_h, z16):
    @functools.partial(
        pl.kernel,
        mesh=_sc_mesh(),
        compiler_params=_SC_PARAMS,
        out_type=[jax.ShapeDtypeStruct((NC, N, DEG_W), jnp.float32),
                  jax.ShapeDtypeStruct((NC, N, DEG_W), jnp.float32)],
        scratch_types=[
            pltpu.VMEM((ROWS_W, EB), jnp.int32),
            pltpu.VMEM((EB, DEG_W), jnp.float32),
            pltpu.VMEM_SHARED((N, DEG_W), jnp.float32),
            pltpu.VMEM_SHARED((N, DEG_W), jnp.float32),
        ],
    )
    def kern(d1h, d2h, ones_hr, z16h, o1, o2, idxv, onesv, t1, t2):
        cid = lax.axis_index("c")
        sid = lax.axis_index("s")
        wid = sid * NC + cid
        r0 = sid * ROWS_T
        pltpu.sync_copy(z16h.at[pl.ds(r0, ROWS_T)], t1.at[pl.ds(r0, ROWS_T)])
        pltpu.sync_copy(z16h.at[pl.ds(r0, ROWS_T)], t2.at[pl.ds(r0, ROWS_T)])
        pltpu.sync_copy(ones_hr, onesv)
        plsc.subcore_barrier()
        for dh, tbl in ((d1h, t1), (d2h, t2)):
            pltpu.sync_copy(dh.at[wid], idxv)

            def body(it, carry, tbl=tbl):
                for j in range(KJ):
                    pltpu.sync_copy(onesv, tbl.at[idxv.at[it * KJ + j]],
                                    add=True)
                return carry
            lax.fori_loop(0, NIT, body, 0)
        plsc.subcore_barrier()
        pltpu.sync_copy(t1.at[pl.ds(r0, ROWS_T)], o1.at[cid, pl.ds(r0, ROWS_T)])
        pltpu.sync_copy(t2.at[pl.ds(r0, ROWS_T)], o2.at[cid, pl.ds(r0, ROWS_T)])

    return kern(d1, d2, ones_h, z16)


def _edge_agg(s1, d1, s2, d2, xs1, xs2, z48):
    @functools.partial(
        pl.kernel,
        mesh=_sc_mesh(),
        compiler_params=_SC_PARAMS,
        out_type=[jax.ShapeDtypeStruct((NC, N, D_PAD), jnp.float32),
                  jax.ShapeDtypeStruct((NC, N, D_PAD), jnp.float32)],
        scratch_types=[
            pltpu.VMEM((ROWS_W, EB), jnp.int32),
            pltpu.VMEM((ROWS_W, EB), jnp.int32),
            pltpu.VMEM((KJ, EB, D_PAD), jnp.float32),
            pltpu.VMEM_SHARED((N, D_PAD), jnp.float32),
            pltpu.VMEM_SHARED((N, D_PAD), jnp.float32),
            pltpu.SemaphoreType.DMA,
        ],
    )
    def kern(s1h, d1h, s2h, d2h, x1h, x2h, z48h, o1, o2,
             sv, dv, rowsv, a1, a2, sem):
        cid = lax.axis_index("c")
        sid = lax.axis_index("s")
        wid = sid * NC + cid
        r0 = sid * ROWS_T
        pltpu.sync_copy(z48h.at[pl.ds(r0, ROWS_T)], a1.at[pl.ds(r0, ROWS_T)])
        pltpu.sync_copy(z48h.at[pl.ds(r0, ROWS_T)], a2.at[pl.ds(r0, ROWS_T)])
        plsc.subcore_barrier()
        for sh, dh, xh, tbl in ((s1h, d1h, x1h, a1), (s2h, d2h, x2h, a2)):
            pltpu.sync_copy(sh.at[wid], sv)
            pltpu.sync_copy(dh.at[wid], dv)

            def body(it, carry, xh=xh, tbl=tbl):
                cps = [pltpu.async_copy(xh.at[sv.at[it * KJ + j]],
                                        rowsv.at[j], sem)
                       for j in range(KJ)]
                for c in cps:
                    c.wait()
                for j in range(KJ):
                    pltpu.sync_copy(rowsv.at[j], tbl.at[dv.at[it * KJ + j]],
                                    add=True)
                return carry
            lax.fori_loop(0, NIT, body, 0)
        plsc.subcore_barrier()
        pltpu.sync_copy(a1.at[pl.ds(r0, ROWS_T)], o1.at[cid, pl.ds(r0, ROWS_T)])
        pltpu.sync_copy(a2.at[pl.ds(r0, ROWS_T)], o2.at[cid, pl.ds(r0, ROWS_T)])

    return kern(s1, d1, s2, d2, xs1, xs2, z48)


def _enc_body(xc_ref, y_ref, d1_ref, d2_ref, wd_ref, bd_ref, fcw_ref, fcb_ref,
              l1w_ref, l1b_ref, wih_ref, whh_ref, bih_ref, bhh_ref,
              g1w_ref, g2w_ref, xs1_ref, xs2_ref, di1_ref, di2_ref):
    xc = xc_ref[...].reshape(5 * BN, 392)
    co = jnp.maximum(
        jnp.dot(xc, wd_ref[...], preferred_element_type=jnp.float32)
        + bd_ref[...], 0.0)
    f = jnp.dot(co, fcw_ref[...], preferred_element_type=jnp.float32) + fcb_ref[...]
    l = jnp.maximum(
        jnp.dot(f, l1w_ref[...], preferred_element_type=jnp.float32)
        + l1b_ref[...], 0.0)
    wih = wih_ref[...]
    whh = whh_ref[...]
    bih = bih_ref[...]
    bhh = bhh_ref[...]
    h = jnp.zeros((BN, 64), jnp.float32)
    for t in range(4):
        xt = jnp.concatenate([l[t * BN:(t + 1) * BN], y_ref[t]], axis=1)
        gi = jnp.dot(xt, wih, preferred_element_type=jnp.float32) + bih
        gh = jnp.dot(h, whh, preferred_element_type=jnp.float32) + bhh
        r = jax.nn.sigmoid(gi[:, 0:64] + gh[:, 0:64])
        z = jax.nn.sigmoid(gi[:, 64:128] + gh[:, 64:128])
        n = jnp.tanh(gi[:, 128:192] + r * gh[:, 128:192])
        h = (1.0 - z) * n + z * h
    hh = jnp.concatenate([l[4 * BN:5 * BN], h], axis=1)
    pad = jnp.zeros((BN, D_PAD - 40), jnp.float32)
    for d_ref, gw_ref, xs_ref, di_ref in (
            (d1_ref, g1w_ref, xs1_ref, di1_ref),
            (d2_ref, g2w_ref, xs2_ref, di2_ref)):
        deg = d_ref[0, :, 0:1] + d_ref[1, :, 0:1] + 1.0
        dinv = lax.rsqrt(deg)
        xw = jnp.dot(hh, gw_ref[...], preferred_element_type=jnp.float32)
        xs_ref[...] = jnp.concatenate([xw * dinv, pad], axis=1)
        di_ref[...] = dinv


def _encoder(xcT, yT, dp1, dp2, wd, bd, fcp, fcb, l1w, l1b,
             wihT, whhT, bih, bhh, g1w, g2w):
    full = lambda shape: pl.BlockSpec(shape, lambda i: tuple(0 for _ in shape))
    return pl.pallas_call(
        _enc_body,
        grid=(N // BN,),
        in_specs=[
            pl.BlockSpec((5, BN, 392), lambda i: (0, i, 0)),
            pl.BlockSpec((5, BN, 1), lambda i: (0, i, 0)),
            pl.BlockSpec((NC, BN, DEG_W), lambda i: (0, i, 0)),
            pl.BlockSpec((NC, BN, DEG_W), lambda i: (0, i, 0)),
            full((392, 784)),
            full((1, 784)),
            full((784, 80)),
            full((1, 80)),
            full((80, 40)),
            full((1, 40)),
            full((41, 192)),
            full((64, 192)),
            full((1, 192)),
            full((1, 192)),
            full((104, 40)),
            full((104, 40)),
        ],
        out_specs=[
            pl.BlockSpec((BN, D_PAD), lambda i: (i, 0)),
            pl.BlockSpec((BN, D_PAD), lambda i: (i, 0)),
            pl.BlockSpec((BN, 1), lambda i: (i, 0)),
            pl.BlockSpec((BN, 1), lambda i: (i, 0)),
        ],
        out_shape=[
            jax.ShapeDtypeStruct((N, D_PAD), jnp.float32),
            jax.ShapeDtypeStruct((N, D_PAD), jnp.float32),
            jax.ShapeDtypeStruct((N, 1), jnp.float32),
            jax.ShapeDtypeStruct((N, 1), jnp.float32),
        ],
    )(xcT, yT, dp1, dp2, wd, bd, fcp, fcb, l1w, l1b,
      wihT, whhT, bih, bhh, g1w, g2w)


def _comb_body(a1_ref, a2_ref, xs1_ref, xs2_ref, di1_ref, di2_ref,
               b1_ref, b2_ref, mw1_ref, mw2_ref, mb_ref, out_ref):
    g1 = jnp.maximum(
        (a1_ref[0] + a1_ref[1] + xs1_ref[...])[:, :40] * di1_ref[...]
        + b1_ref[...], 0.0)
    g2 = jnp.maximum(
        (a2_ref[0] + a2_ref[1] + xs2_ref[...])[:, :40] * di2_ref[...]
        + b2_ref[...], 0.0)
    out_ref[...] = (
        jnp.dot(g1, mw1_ref[...], preferred_element_type=jnp.float32)
        + jnp.dot(g2, mw2_ref[...], preferred_element_type=jnp.float32)
        + mb_ref[...])


def _combine(a1, a2, xs1, xs2, di1, di2, b1, b2, mw1, mw2, mb):
    return pl.pallas_call(
        _comb_body,
        out_shape=jax.ShapeDtypeStruct((N, 1), jnp.float32),
    )(a1, a2, xs1, xs2, di1, di2, b1, b2, mw1, mw2, mb)


def kernel(x, edge_index, feat_edge_index, conv_w, conv_b, fc_w, fc_b,
           lin1_w, lin1_b, gru_w_ih, gru_w_hh, gru_b_ih, gru_b_hh,
           gcn1_w, gcn1_b, gcn2_w, gcn2_b, mlp_w, mlp_b):
    f32 = jnp.float32
    xcT = x[:, :, 3:].transpose(1, 0, 2)
    yT = x[:, :, 2].T[:, :, None]
    wd = jnp.einsum("pq,ck->pkqc", jnp.eye(49, dtype=f32),
                    conv_w[:, 0, :]).reshape(392, 784)
    bd = jnp.tile(conv_b, 49)[None, :]
    fcp = fc_w.reshape(16, 49, 80).transpose(1, 0, 2).reshape(784, 80)

    s1 = edge_index[0].reshape(NW, ROWS_W, EB)
    d1 = edge_index[1].reshape(NW, ROWS_W, EB)
    s2 = feat_edge_index[0].reshape(NW, ROWS_W, EB)
    d2 = feat_edge_index[1].reshape(NW, ROWS_W, EB)
    ones_h = jnp.ones((EB, DEG_W), f32)
    z16 = jnp.zeros((N, DEG_W), f32)
    z48 = jnp.zeros((N, D_PAD), f32)

    dp1, dp2 = _deg_partials(d1, d2, ones_h, z16)
    xs1, xs2, di1, di2 = _encoder(
        xcT, yT, dp1, dp2, wd, bd, fcp, fc_b[None], lin1_w, lin1_b[None],
        gru_w_ih.T, gru_w_hh.T, gru_b_ih[None], gru_b_hh[None], gcn1_w, gcn2_w)
    a1, a2 = _edge_agg(s1, d1, s2, d2, xs1, xs2, z48)
    return _combine(a1, a2, xs1, xs2, di1, di2, gcn1_b[None], gcn2_b[None],
                    mlp_w[:40], mlp_w[40:], mlp_b[None])
